# Initial kernel scaffold; baseline (speedup 1.0000x reference)
import jax
import jax.numpy as jnp
from jax.experimental import pallas as pl


def kernel(x, batch, positions, W1, b1, W2, b2):
    raise NotImplementedError("write your pallas kernel here")


# speedup vs baseline: 11.1971x; 11.1971x over previous
import functools

import jax
import jax.numpy as jnp
from jax import lax
from jax.experimental import pallas as pl
from jax.experimental.pallas import tpu as pltpu

N = 100000
D = 8
PROJ = 64
POS_DIM = 3
NUM_SEGMENTS = 512
BLK = 2000


def _pool_body(pos_ref, x_ref, ids_ref, w1_ref, b1_ref, w2_ref, b2_ref, out_ref):
    i = pl.program_id(0)

    pos = pos_ref[...]
    h = jnp.dot(pos, w1_ref[...], preferred_element_type=jnp.float32) + b1_ref[...]
    h = jnp.where(h > 0, h, jnp.exp(h) - 1.0)
    w = jnp.dot(h, w2_ref[...], preferred_element_type=jnp.float32) + b2_ref[...]
    w = w - jnp.max(w, axis=-1, keepdims=True)
    w = jnp.exp(w)
    w = w / jnp.sum(w, axis=-1, keepdims=True)

    ids = ids_ref[0, 0, :]
    seg = lax.broadcasted_iota(jnp.int32, (BLK, NUM_SEGMENTS), 1)
    onehot = (ids[:, None] == seg).astype(jnp.float32)

    @pl.when(i == 0)
    def _init():
        out_ref[...] = jnp.zeros_like(out_ref)

    xw = x_ref[...] * w[:, None, :]
    for d in range(D):
        contrib = lax.dot_general(
            onehot, xw[:, d, :],
            dimension_numbers=(((0,), (0,)), ((), ())),
            preferred_element_type=jnp.float32,
        )
        out_ref[d, :, :] += contrib


def kernel(x, batch, positions, W1, b1, W2, b2):
    nblocks = N // BLK
    ids = batch.astype(jnp.int32).reshape(nblocks, 1, BLK)
    pooled = pl.pallas_call(
        _pool_body,
        grid=(nblocks,),
        in_specs=[
            pl.BlockSpec((BLK, POS_DIM), lambda i: (i, 0)),
            pl.BlockSpec((BLK, D, PROJ), lambda i: (i, 0, 0)),
            pl.BlockSpec((1, 1, BLK), lambda i: (i, 0, 0)),
            pl.BlockSpec((POS_DIM, PROJ), lambda i: (0, 0)),
            pl.BlockSpec((1, PROJ), lambda i: (0, 0)),
            pl.BlockSpec((PROJ, PROJ), lambda i: (0, 0)),
            pl.BlockSpec((1, PROJ), lambda i: (0, 0)),
        ],
        out_specs=pl.BlockSpec((D, NUM_SEGMENTS, PROJ), lambda i: (0, 0, 0)),
        out_shape=jax.ShapeDtypeStruct((D, NUM_SEGMENTS, PROJ), jnp.float32),
    )(positions, x, ids, W1, b1.reshape(1, PROJ), W2, b2.reshape(1, PROJ))
    return jnp.transpose(pooled, (1, 2, 0))[..., None]
